# baseline (speedup 1.0000x reference)
import jax
import jax.numpy as jnp
from jax.experimental import pallas as pl
from jax.experimental.pallas import tpu as pltpu

_P, _A, _S = 2048, 2048, 512
_N = _P + _A + _S
_D = 128
_H = 64
_TH = 0.2
_T = 256
_PT = 256

_f32 = jnp.float32
_bf16 = jnp.bfloat16


def _dot(a, b):
    return jax.lax.dot_general(a.astype(_bf16), b.astype(_bf16),
                               (((1,), (0,)), ((), ())),
                               preferred_element_type=_f32)


def _dot_nt(a, b):
    return jax.lax.dot_general(a.astype(_bf16), b.astype(_bf16),
                               (((1,), (1,)), ((), ())),
                               preferred_element_type=_f32)


def _unorm(x, m):
    y = x * m
    n = jnp.sqrt(jnp.sum(y * y, axis=1, keepdims=True))
    return y / jnp.maximum(n, 1e-8)


def _u2(x, mw):
    return jnp.concatenate(
        [_unorm(x, mw[0:1]), _unorm(x, mw[1:2])], axis=1).astype(_bf16)


def _thresh(g):
    s = 0.5 * g
    return jnp.where(s < _TH, jnp.zeros_like(s), s)


def _sm(v):
    e = jnp.exp(v - jnp.max(v, axis=1, keepdims=True))
    return e / jnp.sum(e, axis=1, keepdims=True)


def _prep_u_body(fp, fa, fs, mp1, mp2, fgg, sg1, sg2,
                 ut, u1, u2, ura, urs):
    ut[...] = _u2(fp[...], fgg[...])
    u1[...] = _u2(mp1[...], sg1[...])
    u2[...] = _u2(mp2[...], sg2[...])
    ura[...] = _u2(fa[...], fgg[...])
    urs[...] = _u2(fs[...], fgg[...])


def _sw_body(u_t, u, w, out):
    m = _thresh(_dot_nt(u_t[...], u[...]))
    out[...] = _dot(m, w[...])


def _prop_body(adj_a, adj_s, fa, fs, swa, sws, fp_t, gw1,
               fpw, tpaw, tpsw, bpa, bps,
               ufa, ufs, usa, uss, h1):
    aa = adj_a[...].astype(_bf16)
    as_ = adj_s[...].astype(_bf16)
    prop_a = jax.lax.dot_general(aa, fa[...].astype(_bf16),
                                 (((1,), (0,)), ((), ())),
                                 preferred_element_type=_f32)
    ufa[...] = _u2(prop_a, fpw[...])
    prop_s = jax.lax.dot_general(as_, fs[...].astype(_bf16),
                                 (((1,), (0,)), ((), ())),
                                 preferred_element_type=_f32)
    ufs[...] = _u2(prop_s, fpw[...])
    hid_a = jax.lax.dot_general(aa, swa[...].astype(_bf16),
                                (((1,), (0,)), ((), ())),
                                preferred_element_type=_f32) + bpa[...]
    usa[...] = _u2(hid_a, tpaw[...])
    hid_s = jax.lax.dot_general(as_, sws[...].astype(_bf16),
                                (((1,), (0,)), ((), ())),
                                preferred_element_type=_f32) + bps[...]
    uss[...] = _u2(hid_s, tpsw[...])
    h1[...] = _dot(fp_t[...], gw1[...])


def _colsum_body(*refs):
    us_i = refs[0:7]
    us_j = refs[7:14]
    c = refs[14]
    ccol = refs[15]
    crow = refs[16]
    i = pl.program_id(0)
    j = pl.program_id(1)
    nj = pl.num_programs(1)

    @pl.when((i == 0) & (j == 0))
    def _():
        ccol[...] = jnp.zeros((8, _P), _f32)
        crow[...] = jnp.zeros((_P, 8), _f32)

    @pl.when(j >= i)
    def _():
        for k in range(7):
            m = _thresh(_dot_nt(us_i[k][...], us_j[k][...]))
            ccol[k:k + 1, pl.ds(j * _T, _T)] += jnp.sum(m, axis=0,
                                                        keepdims=True)

            @pl.when(j > i)
            def _():
                crow[pl.ds(i * _T, _T), k:k + 1] += jnp.sum(m, axis=1,
                                                            keepdims=True)

    @pl.when((i == nj - 1) & (j == nj - 1))
    def _():
        c[...] = ccol[...] + crow[...].T


def _beta_body(c, sgw, ffw, fw, beta, betaT):
    sg = _sm(sgw[...])
    ff = _sm(ffw[...])
    f4 = _sm(fw[...])
    cs = [c[k:k + 1, :] for k in range(7)]
    cp = [jnp.maximum(x, 1e-12) for x in cs]
    t = [(x > 0.0).astype(_f32) for x in cs]
    eps = jnp.float32(1e-12)

    e_t = jnp.maximum(t[0], eps)
    e_sem = jnp.maximum(sg[0:1, 0:1] * t[1] + sg[0:1, 1:2] * t[2], eps)
    e_fp = jnp.maximum(ff[0:1, 0:1] * t[3] + ff[0:1, 1:2] * t[4], eps)
    e_st = jnp.maximum(ff[0:1, 0:1] * t[5] + ff[0:1, 1:2] * t[6], eps)

    rows = [
        f4[0:1, 0:1] / (cp[0] * e_t),
        f4[0:1, 1:2] * sg[0:1, 0:1] / (cp[1] * e_sem),
        f4[0:1, 1:2] * sg[0:1, 1:2] / (cp[2] * e_sem),
        f4[0:1, 2:3] * ff[0:1, 0:1] / (cp[3] * e_fp),
        f4[0:1, 2:3] * ff[0:1, 1:2] / (cp[4] * e_fp),
        f4[0:1, 3:4] * ff[0:1, 0:1] / (cp[5] * e_st),
        f4[0:1, 3:4] * ff[0:1, 1:2] / (cp[6] * e_st),
        jnp.zeros((1, _P), _f32),
    ]
    b = jnp.concatenate(rows, axis=0)
    beta[...] = b
    betaT[...] = b.T


def _combine_body(*refs):
    us_i = refs[0:7]
    us_j = refs[7:14]
    beta = refs[14]
    betaT = refs[15]
    up = refs[16]
    lo = refs[17]
    d = refs[18]
    dcol = refs[19]
    drow = refs[20]
    i = pl.program_id(0)
    j = pl.program_id(1)
    nj = pl.num_programs(1)

    @pl.when((i == 0) & (j == 0))
    def _():
        dcol[...] = jnp.zeros((1, _P), _f32)
        drow[...] = jnp.zeros((_P, 1), _f32)

    @pl.when(j >= i)
    def _():
        acc = jnp.zeros((_T, _T), _f32)
        for k in range(7):
            m = _thresh(_dot_nt(us_i[k][...], us_j[k][...]))
            bj = beta[k:k + 1, pl.ds(j * _T, _T)]
            bi = betaT[...][:, k:k + 1]
            acc = acc + m * (bi + bj)
        up[...] = acc
        lo[...] = acc.T
        dcol[0:1, pl.ds(j * _T, _T)] += jnp.sum(acc, axis=0, keepdims=True)

        @pl.when(j > i)
        def _():
            drow[pl.ds(i * _T, _T), 0:1] += jnp.sum(acc, axis=1,
                                                    keepdims=True)

    @pl.when((i == nj - 1) & (j == nj - 1))
    def _():
        d[...] = dcol[...] + drow[...].T


def _fin1_body(up, lo, d, h1, b1, adj, x1):
    i = pl.program_id(0)
    colid = jax.lax.broadcasted_iota(jnp.int32, (_T, _P), 1)
    un = jnp.where(colid >= i * _T, up[...], lo[...])
    dr = jnp.maximum(d[0:1, :], 1e-12)
    a = un / dr
    adj[...] = a
    x1[...] = jax.nn.relu(_dot(a, h1[...]) + b1[...])


def _fin2_body(adj, x1, w2, b2, out):
    h2 = _dot(x1[...], w2[...])
    y = _dot(adj[...], h2) + b2[...]
    y = y - jnp.max(y, axis=1, keepdims=True)
    out[...] = y - jnp.log(jnp.sum(jnp.exp(y), axis=1, keepdims=True))


def _full(shape):
    nd = len(shape)
    return pl.BlockSpec(shape, lambda i: (0,) * nd)


def kernel(features, adj_ori, mp_emb_mp1, mp_emb_mp2, enc_W, enc_b,
           fgg_origin_w, fp_origin_w, sgg_w, fgg_topo_pa_w, fgg_topo_ps_w,
           W_topo_pa, b_topo_pa, W_topo_ps, b_topo_ps, sg_agg_w, f_agg_f_w,
           f_agg_w, gcn_W1, gcn_b1, gcn_W2, gcn_b2):
    f_p = features[0:_P]
    f_a = features[_P:_P + _A]
    f_s = features[_P + _A:_N]
    mp1 = mp_emb_mp1[0:_P]
    mp2 = mp_emb_mp2[0:_P]
    sg1 = sgg_w[0]
    sg2 = sgg_w[1]
    bpa = b_topo_pa.reshape(1, _H)
    bps = b_topo_ps.reshape(1, _H)
    b1 = gcn_b1.reshape(1, _H)
    b2 = gcn_b2.reshape(1, 4)
    sgw = sg_agg_w.reshape(1, 2)
    ffw = f_agg_f_w.reshape(1, 2)
    fw = f_agg_w.reshape(1, 4)

    sd = jax.ShapeDtypeStruct
    u_t, u_1, u_2, u_ra, u_rs = pl.pallas_call(
        _prep_u_body,
        grid=(1,),
        in_specs=[_full((_P, _D)), _full((_A, _D)), _full((_S, _D)),
                  _full((_P, _H)), _full((_P, _H)), _full((2, _D)),
                  _full((2, _H)), _full((2, _H))],
        out_specs=[_full((_P, 2 * _D)), _full((_P, _D)), _full((_P, _D)),
                   _full((_A, 2 * _D)), _full((_S, 2 * _D))],
        out_shape=[sd((_P, 2 * _D), _bf16), sd((_P, _D), _bf16),
                   sd((_P, _D), _bf16), sd((_A, 2 * _D), _bf16),
                   sd((_S, 2 * _D), _bf16)],
    )(f_p, f_a, f_s, mp1, mp2, fgg_origin_w, sg1, sg2)

    sw_a = pl.pallas_call(
        _sw_body,
        grid=(_A // _T,),
        in_specs=[pl.BlockSpec((_T, 2 * _D), lambda i: (i, 0)),
                  _full((_A, 2 * _D)), _full((_A, _H))],
        out_specs=pl.BlockSpec((_T, _H), lambda i: (i, 0)),
        out_shape=sd((_A, _H), _f32),
    )(u_ra, u_ra, W_topo_pa)

    sw_s = pl.pallas_call(
        _sw_body,
        grid=(1,),
        in_specs=[_full((_S, 2 * _D)), _full((_S, 2 * _D)), _full((_S, _H))],
        out_specs=_full((_S, _H)),
        out_shape=sd((_S, _H), _f32),
    )(u_rs, u_rs, W_topo_ps)

    (u_fa, u_fs, u_sa, u_ss, h1) = pl.pallas_call(
        _prop_body,
        grid=(_P // _PT,),
        in_specs=[
            pl.BlockSpec((_PT, _A), lambda i: (i, _P // _A)),
            pl.BlockSpec((_PT, _S), lambda i: (i, (_P + _A) // _S)),
            _full((_A, _D)), _full((_S, _D)),
            _full((_A, _H)), _full((_S, _H)),
            pl.BlockSpec((_PT, _D), lambda i: (i, 0)),
            _full((_D, _H)),
            _full((2, _D)), _full((2, _H)), _full((2, _H)),
            _full((1, _H)), _full((1, _H)),
        ],
        out_specs=[
            pl.BlockSpec((_PT, 2 * _D), lambda i: (i, 0)),
            pl.BlockSpec((_PT, 2 * _D), lambda i: (i, 0)),
            pl.BlockSpec((_PT, _D), lambda i: (i, 0)),
            pl.BlockSpec((_PT, _D), lambda i: (i, 0)),
            pl.BlockSpec((_PT, _H), lambda i: (i, 0)),
        ],
        out_shape=[sd((_P, 2 * _D), _bf16), sd((_P, 2 * _D), _bf16),
                   sd((_P, _D), _bf16), sd((_P, _D), _bf16),
                   sd((_P, _H), _f32)],
    )(adj_ori, adj_ori, f_a, f_s, sw_a, sw_s, f_p, gcn_W1,
      fp_origin_w, fgg_topo_pa_w, fgg_topo_ps_w, bpa, bps)

    us = [u_t, u_1, u_2, u_fa, u_fs, u_sa, u_ss]
    ui_specs = [pl.BlockSpec((_T, u.shape[1]), lambda i, j: (i, 0))
                for u in us]
    uj_specs = [pl.BlockSpec((_T, u.shape[1]), lambda i, j: (j, 0))
                for u in us]
    nb = _P // _T

    c = pl.pallas_call(
        _colsum_body,
        grid=(nb, nb),
        in_specs=ui_specs + uj_specs,
        out_specs=pl.BlockSpec((8, _P), lambda i, j: (0, 0)),
        out_shape=sd((8, _P), _f32),
        scratch_shapes=[pltpu.VMEM((8, _P), _f32),
                        pltpu.VMEM((_P, 8), _f32)],
    )(*us, *us)

    beta, betaT = pl.pallas_call(
        _beta_body,
        grid=(1,),
        in_specs=[_full((8, _P)), _full((1, 2)), _full((1, 2)), _full((1, 4))],
        out_specs=[_full((8, _P)), _full((_P, 8))],
        out_shape=[sd((8, _P), _f32), sd((_P, 8), _f32)],
    )(c, sgw, ffw, fw)

    un_up, un_lo, dvec = pl.pallas_call(
        _combine_body,
        grid=(nb, nb),
        in_specs=ui_specs + uj_specs + [
            pl.BlockSpec((8, _P), lambda i, j: (0, 0)),
            pl.BlockSpec((_T, 8), lambda i, j: (i, 0))],
        out_specs=[pl.BlockSpec((_T, _T), lambda i, j: (i, j)),
                   pl.BlockSpec((_T, _T), lambda i, j: (j, i)),
                   pl.BlockSpec((1, _P), lambda i, j: (0, 0))],
        out_shape=[sd((_P, _P), _f32), sd((_P, _P), _f32),
                   sd((1, _P), _f32)],
        scratch_shapes=[pltpu.VMEM((1, _P), _f32),
                        pltpu.VMEM((_P, 1), _f32)],
    )(*us, *us, beta, betaT)

    new_adj, x1 = pl.pallas_call(
        _fin1_body,
        grid=(_P // _T,),
        in_specs=[pl.BlockSpec((_T, _P), lambda i: (i, 0)),
                  pl.BlockSpec((_T, _P), lambda i: (i, 0)),
                  _full((1, _P)), _full((_P, _H)), _full((1, _H))],
        out_specs=[pl.BlockSpec((_T, _P), lambda i: (i, 0)),
                   pl.BlockSpec((_T, _H), lambda i: (i, 0))],
        out_shape=[sd((_P, _P), _f32), sd((_P, _H), _f32)],
    )(un_up, un_lo, dvec, h1, b1)

    logits = pl.pallas_call(
        _fin2_body,
        grid=(_P // _T,),
        in_specs=[pl.BlockSpec((_T, _P), lambda i: (i, 0)), _full((_P, _H)),
                  _full((_H, 4)), _full((1, 4))],
        out_specs=pl.BlockSpec((_T, 4), lambda i: (i, 0)),
        out_shape=sd((_P, 4), _f32),
    )(new_adj, x1, gcn_W2, b2)

    return logits, new_adj


# speedup vs baseline: 1.8627x; 1.8627x over previous
import jax
import jax.numpy as jnp
from jax.experimental import pallas as pl
from jax.experimental.pallas import tpu as pltpu

_P, _A, _S = 2048, 2048, 512
_N = _P + _A + _S
_D = 128
_H = 64
_TH = 0.2
_T = 256
_PT = 256

_f32 = jnp.float32
_bf16 = jnp.bfloat16


def _dot(a, b):
    return jax.lax.dot_general(a.astype(_bf16), b.astype(_bf16),
                               (((1,), (0,)), ((), ())),
                               preferred_element_type=_f32)


def _dot_nt(a, b):
    return jax.lax.dot_general(a.astype(_bf16), b.astype(_bf16),
                               (((1,), (1,)), ((), ())),
                               preferred_element_type=_f32)


def _unorm(x, m):
    y = x * m
    n = jnp.sqrt(jnp.sum(y * y, axis=1, keepdims=True))
    return y / jnp.maximum(n, 1e-8)


def _u2(x, mw):
    return jnp.concatenate(
        [_unorm(x, mw[0:1]), _unorm(x, mw[1:2])], axis=1).astype(_bf16)


def _thresh(g):
    s = 0.5 * g
    return jnp.where(s < _TH, jnp.zeros_like(s), s)


def _sm(v):
    e = jnp.exp(v - jnp.max(v, axis=1, keepdims=True))
    return e / jnp.sum(e, axis=1, keepdims=True)


def _prep_u_body(fp, fa, fs, mp1, mp2, fgg, sg1, sg2,
                 ut, u1, u2, ura, urs):
    ut[...] = _u2(fp[...], fgg[...])
    u1[...] = _u2(mp1[...], sg1[...])
    u2[...] = _u2(mp2[...], sg2[...])
    ura[...] = _u2(fa[...], fgg[...])
    urs[...] = _u2(fs[...], fgg[...])


def _sw_body(u_t, u, w, out):
    m = _thresh(_dot_nt(u_t[...], u[...]))
    out[...] = _dot(m, w[...])


def _prop_body(adj_a, adj_s, fa, fs, swa, sws, fp_t, gw1,
               fpw, tpaw, tpsw, bpa, bps,
               ufa, ufs, usa, uss, h1):
    aa = adj_a[...].astype(_bf16)
    as_ = adj_s[...].astype(_bf16)
    prop_a = jax.lax.dot_general(aa, fa[...].astype(_bf16),
                                 (((1,), (0,)), ((), ())),
                                 preferred_element_type=_f32)
    ufa[...] = _u2(prop_a, fpw[...])
    prop_s = jax.lax.dot_general(as_, fs[...].astype(_bf16),
                                 (((1,), (0,)), ((), ())),
                                 preferred_element_type=_f32)
    ufs[...] = _u2(prop_s, fpw[...])
    hid_a = jax.lax.dot_general(aa, swa[...].astype(_bf16),
                                (((1,), (0,)), ((), ())),
                                preferred_element_type=_f32) + bpa[...]
    usa[...] = _u2(hid_a, tpaw[...])
    hid_s = jax.lax.dot_general(as_, sws[...].astype(_bf16),
                                (((1,), (0,)), ((), ())),
                                preferred_element_type=_f32) + bps[...]
    uss[...] = _u2(hid_s, tpsw[...])
    h1[...] = _dot(fp_t[...], gw1[...])


def _colsum_body(*refs):
    us_t = refs[0:7]
    us = refs[7:14]
    c = refs[14]
    i = pl.program_id(0)
    rows = []
    for u_t, u in zip(us_t, us):
        m = _thresh(_dot_nt(u_t[...], u[...]))
        rows.append(jnp.sum(m, axis=0, keepdims=True))
    rows.append(jnp.zeros((1, _P), _f32))
    blk = jnp.concatenate(rows, axis=0)

    @pl.when(i == 0)
    def _():
        c[...] = blk

    @pl.when(i > 0)
    def _():
        c[...] = c[...] + blk


def _beta_body(c, sgw, ffw, fw, beta, betaT):
    sg = _sm(sgw[...])
    ff = _sm(ffw[...])
    f4 = _sm(fw[...])
    cs = [c[k:k + 1, :] for k in range(7)]
    cp = [jnp.maximum(x, 1e-12) for x in cs]
    t = [(x > 0.0).astype(_f32) for x in cs]
    eps = jnp.float32(1e-12)

    e_t = jnp.maximum(t[0], eps)
    e_sem = jnp.maximum(sg[0:1, 0:1] * t[1] + sg[0:1, 1:2] * t[2], eps)
    e_fp = jnp.maximum(ff[0:1, 0:1] * t[3] + ff[0:1, 1:2] * t[4], eps)
    e_st = jnp.maximum(ff[0:1, 0:1] * t[5] + ff[0:1, 1:2] * t[6], eps)

    rows = [
        f4[0:1, 0:1] / (cp[0] * e_t),
        f4[0:1, 1:2] * sg[0:1, 0:1] / (cp[1] * e_sem),
        f4[0:1, 1:2] * sg[0:1, 1:2] / (cp[2] * e_sem),
        f4[0:1, 2:3] * ff[0:1, 0:1] / (cp[3] * e_fp),
        f4[0:1, 2:3] * ff[0:1, 1:2] / (cp[4] * e_fp),
        f4[0:1, 3:4] * ff[0:1, 0:1] / (cp[5] * e_st),
        f4[0:1, 3:4] * ff[0:1, 1:2] / (cp[6] * e_st),
        jnp.zeros((1, _P), _f32),
    ]
    b = jnp.concatenate(rows, axis=0)
    beta[...] = b
    betaT[...] = b.T


def _combine_body(*refs):
    us_t = refs[0:7]
    us = refs[7:14]
    beta = refs[14]
    betaT = refs[15]
    unnorm = refs[16]
    d = refs[17]
    i = pl.program_id(0)
    bT = betaT[...]
    acc = jnp.zeros((_T, _P), _f32)
    for k, (u_t, u) in enumerate(zip(us_t, us)):
        m = _thresh(_dot_nt(u_t[...], u[...]))
        bj = beta[k:k + 1, :]
        bi = bT[:, k:k + 1]
        acc = acc + m * (bi + bj)
    unnorm[...] = acc
    dpart = jnp.concatenate(
        [jnp.sum(acc, axis=0, keepdims=True), jnp.zeros((7, _P), _f32)], axis=0)

    @pl.when(i == 0)
    def _():
        d[...] = dpart

    @pl.when(i > 0)
    def _():
        d[...] = d[...] + dpart


def _fin1_body(un, d, h1, b1, adj, x1):
    dr = jnp.maximum(d[0:1, :], 1e-12)
    a = un[...] / dr
    adj[...] = a
    x1[...] = jax.nn.relu(_dot(a, h1[...]) + b1[...])


def _fin2_body(adj, x1, w2, b2, out):
    h2 = _dot(x1[...], w2[...])
    y = _dot(adj[...], h2) + b2[...]
    y = y - jnp.max(y, axis=1, keepdims=True)
    out[...] = y - jnp.log(jnp.sum(jnp.exp(y), axis=1, keepdims=True))


def _full(shape):
    nd = len(shape)
    return pl.BlockSpec(shape, lambda i: (0,) * nd)


def kernel(features, adj_ori, mp_emb_mp1, mp_emb_mp2, enc_W, enc_b,
           fgg_origin_w, fp_origin_w, sgg_w, fgg_topo_pa_w, fgg_topo_ps_w,
           W_topo_pa, b_topo_pa, W_topo_ps, b_topo_ps, sg_agg_w, f_agg_f_w,
           f_agg_w, gcn_W1, gcn_b1, gcn_W2, gcn_b2):
    f_p = features[0:_P]
    f_a = features[_P:_P + _A]
    f_s = features[_P + _A:_N]
    mp1 = mp_emb_mp1[0:_P]
    mp2 = mp_emb_mp2[0:_P]
    sg1 = sgg_w[0]
    sg2 = sgg_w[1]
    bpa = b_topo_pa.reshape(1, _H)
    bps = b_topo_ps.reshape(1, _H)
    b1 = gcn_b1.reshape(1, _H)
    b2 = gcn_b2.reshape(1, 4)
    sgw = sg_agg_w.reshape(1, 2)
    ffw = f_agg_f_w.reshape(1, 2)
    fw = f_agg_w.reshape(1, 4)

    sd = jax.ShapeDtypeStruct
    u_t, u_1, u_2, u_ra, u_rs = pl.pallas_call(
        _prep_u_body,
        grid=(1,),
        in_specs=[_full((_P, _D)), _full((_A, _D)), _full((_S, _D)),
                  _full((_P, _H)), _full((_P, _H)), _full((2, _D)),
                  _full((2, _H)), _full((2, _H))],
        out_specs=[_full((_P, 2 * _D)), _full((_P, _D)), _full((_P, _D)),
                   _full((_A, 2 * _D)), _full((_S, 2 * _D))],
        out_shape=[sd((_P, 2 * _D), _bf16), sd((_P, _D), _bf16),
                   sd((_P, _D), _bf16), sd((_A, 2 * _D), _bf16),
                   sd((_S, 2 * _D), _bf16)],
    )(f_p, f_a, f_s, mp1, mp2, fgg_origin_w, sg1, sg2)

    sw_a = pl.pallas_call(
        _sw_body,
        grid=(_A // _T,),
        in_specs=[pl.BlockSpec((_T, 2 * _D), lambda i: (i, 0)),
                  _full((_A, 2 * _D)), _full((_A, _H))],
        out_specs=pl.BlockSpec((_T, _H), lambda i: (i, 0)),
        out_shape=sd((_A, _H), _f32),
    )(u_ra, u_ra, W_topo_pa)

    sw_s = pl.pallas_call(
        _sw_body,
        grid=(1,),
        in_specs=[_full((_S, 2 * _D)), _full((_S, 2 * _D)), _full((_S, _H))],
        out_specs=_full((_S, _H)),
        out_shape=sd((_S, _H), _f32),
    )(u_rs, u_rs, W_topo_ps)

    (u_fa, u_fs, u_sa, u_ss, h1) = pl.pallas_call(
        _prop_body,
        grid=(_P // _PT,),
        in_specs=[
            pl.BlockSpec((_PT, _A), lambda i: (i, _P // _A)),
            pl.BlockSpec((_PT, _S), lambda i: (i, (_P + _A) // _S)),
            _full((_A, _D)), _full((_S, _D)),
            _full((_A, _H)), _full((_S, _H)),
            pl.BlockSpec((_PT, _D), lambda i: (i, 0)),
            _full((_D, _H)),
            _full((2, _D)), _full((2, _H)), _full((2, _H)),
            _full((1, _H)), _full((1, _H)),
        ],
        out_specs=[
            pl.BlockSpec((_PT, 2 * _D), lambda i: (i, 0)),
            pl.BlockSpec((_PT, 2 * _D), lambda i: (i, 0)),
            pl.BlockSpec((_PT, _D), lambda i: (i, 0)),
            pl.BlockSpec((_PT, _D), lambda i: (i, 0)),
            pl.BlockSpec((_PT, _H), lambda i: (i, 0)),
        ],
        out_shape=[sd((_P, 2 * _D), _bf16), sd((_P, 2 * _D), _bf16),
                   sd((_P, _D), _bf16), sd((_P, _D), _bf16),
                   sd((_P, _H), _f32)],
    )(adj_ori, adj_ori, f_a, f_s, sw_a, sw_s, f_p, gcn_W1,
      fp_origin_w, fgg_topo_pa_w, fgg_topo_ps_w, bpa, bps)

    us = [u_t, u_1, u_2, u_fa, u_fs, u_sa, u_ss]
    ut_specs = [pl.BlockSpec((_T, u.shape[1]), lambda i: (i, 0)) for u in us]
    uf_specs = [_full(u.shape) for u in us]

    c = pl.pallas_call(
        _colsum_body,
        grid=(_P // _T,),
        in_specs=ut_specs + uf_specs,
        out_specs=_full((8, _P)),
        out_shape=sd((8, _P), _f32),
    )(*us, *us)

    beta, betaT = pl.pallas_call(
        _beta_body,
        grid=(1,),
        in_specs=[_full((8, _P)), _full((1, 2)), _full((1, 2)), _full((1, 4))],
        out_specs=[_full((8, _P)), _full((_P, 8))],
        out_shape=[sd((8, _P), _f32), sd((_P, 8), _f32)],
    )(c, sgw, ffw, fw)

    unnorm, dvec = pl.pallas_call(
        _combine_body,
        grid=(_P // _T,),
        in_specs=ut_specs + uf_specs + [
            _full((8, _P)), pl.BlockSpec((_T, 8), lambda i: (i, 0))],
        out_specs=[pl.BlockSpec((_T, _P), lambda i: (i, 0)), _full((8, _P))],
        out_shape=[sd((_P, _P), _f32), sd((8, _P), _f32)],
    )(*us, *us, beta, betaT)

    new_adj, x1 = pl.pallas_call(
        _fin1_body,
        grid=(_P // _T,),
        in_specs=[pl.BlockSpec((_T, _P), lambda i: (i, 0)), _full((8, _P)),
                  _full((_P, _H)), _full((1, _H))],
        out_specs=[pl.BlockSpec((_T, _P), lambda i: (i, 0)),
                   pl.BlockSpec((_T, _H), lambda i: (i, 0))],
        out_shape=[sd((_P, _P), _f32), sd((_P, _H), _f32)],
    )(unnorm, dvec, h1, b1)

    logits = pl.pallas_call(
        _fin2_body,
        grid=(_P // _T,),
        in_specs=[pl.BlockSpec((_T, _P), lambda i: (i, 0)), _full((_P, _H)),
                  _full((_H, 4)), _full((1, 4))],
        out_specs=pl.BlockSpec((_T, 4), lambda i: (i, 0)),
        out_shape=sd((_P, 4), _f32),
    )(new_adj, x1, gcn_W2, b2)

    return logits, new_adj


# speedup vs baseline: 1.9965x; 1.0718x over previous
import jax
import jax.numpy as jnp
from jax.experimental import pallas as pl
from jax.experimental.pallas import tpu as pltpu

_P, _A, _S = 2048, 2048, 512
_N = _P + _A + _S
_D = 128
_H = 64
_TH = 0.2
_T = 512
_PT = 256

_f32 = jnp.float32
_bf16 = jnp.bfloat16


def _dot(a, b):
    return jax.lax.dot_general(a.astype(_bf16), b.astype(_bf16),
                               (((1,), (0,)), ((), ())),
                               preferred_element_type=_f32)


def _dot_nt(a, b):
    return jax.lax.dot_general(a.astype(_bf16), b.astype(_bf16),
                               (((1,), (1,)), ((), ())),
                               preferred_element_type=_f32)


def _unorm(x, m):
    y = x * m
    n = jnp.sqrt(jnp.sum(y * y, axis=1, keepdims=True))
    return y / jnp.maximum(n, 1e-8)


def _u2(x, mw):
    return jnp.concatenate(
        [_unorm(x, mw[0:1]), _unorm(x, mw[1:2])], axis=1).astype(_bf16)


def _thresh(g):
    s = 0.5 * g
    return jnp.where(s < _TH, jnp.zeros_like(s), s)


def _sm(v):
    e = jnp.exp(v - jnp.max(v, axis=1, keepdims=True))
    return e / jnp.sum(e, axis=1, keepdims=True)


def _prep_u_body(fp, fa, fs, mp1, mp2, fgg, sg1, sg2,
                 ut, u1, u2, ura, urs):
    ut[...] = _u2(fp[...], fgg[...])
    u1[...] = _u2(mp1[...], sg1[...])
    u2[...] = _u2(mp2[...], sg2[...])
    ura[...] = _u2(fa[...], fgg[...])
    urs[...] = _u2(fs[...], fgg[...])


def _sw_body(u_t, u, w, out):
    m = _thresh(_dot_nt(u_t[...], u[...]))
    out[...] = _dot(m, w[...])


def _prop_body(adj_a, adj_s, fa, fs, swa, sws, fp_t, gw1,
               fpw, tpaw, tpsw, bpa, bps,
               ufa, ufs, usa, uss, h1):
    aa = adj_a[...].astype(_bf16)
    as_ = adj_s[...].astype(_bf16)
    prop_a = jax.lax.dot_general(aa, fa[...].astype(_bf16),
                                 (((1,), (0,)), ((), ())),
                                 preferred_element_type=_f32)
    ufa[...] = _u2(prop_a, fpw[...])
    prop_s = jax.lax.dot_general(as_, fs[...].astype(_bf16),
                                 (((1,), (0,)), ((), ())),
                                 preferred_element_type=_f32)
    ufs[...] = _u2(prop_s, fpw[...])
    hid_a = jax.lax.dot_general(aa, swa[...].astype(_bf16),
                                (((1,), (0,)), ((), ())),
                                preferred_element_type=_f32) + bpa[...]
    usa[...] = _u2(hid_a, tpaw[...])
    hid_s = jax.lax.dot_general(as_, sws[...].astype(_bf16),
                                (((1,), (0,)), ((), ())),
                                preferred_element_type=_f32) + bps[...]
    uss[...] = _u2(hid_s, tpsw[...])
    h1[...] = _dot(fp_t[...], gw1[...])


def _colsum_body(*refs):
    us_t = refs[0:7]
    us = refs[7:14]
    c = refs[14]
    i = pl.program_id(0)
    rows = []
    for u_t, u in zip(us_t, us):
        m = _thresh(_dot_nt(u_t[...], u[...]))
        rows.append(jnp.sum(m, axis=0, keepdims=True))
    rows.append(jnp.zeros((1, _P), _f32))
    blk = jnp.concatenate(rows, axis=0)

    @pl.when(i == 0)
    def _():
        c[...] = blk

    @pl.when(i > 0)
    def _():
        c[...] = c[...] + blk


def _beta_body(c, sgw, ffw, fw, beta, betaT):
    sg = _sm(sgw[...])
    ff = _sm(ffw[...])
    f4 = _sm(fw[...])
    cs = [c[k:k + 1, :] for k in range(7)]
    cp = [jnp.maximum(x, 1e-12) for x in cs]
    t = [(x > 0.0).astype(_f32) for x in cs]
    eps = jnp.float32(1e-12)

    e_t = jnp.maximum(t[0], eps)
    e_sem = jnp.maximum(sg[0:1, 0:1] * t[1] + sg[0:1, 1:2] * t[2], eps)
    e_fp = jnp.maximum(ff[0:1, 0:1] * t[3] + ff[0:1, 1:2] * t[4], eps)
    e_st = jnp.maximum(ff[0:1, 0:1] * t[5] + ff[0:1, 1:2] * t[6], eps)

    rows = [
        f4[0:1, 0:1] / (cp[0] * e_t),
        f4[0:1, 1:2] * sg[0:1, 0:1] / (cp[1] * e_sem),
        f4[0:1, 1:2] * sg[0:1, 1:2] / (cp[2] * e_sem),
        f4[0:1, 2:3] * ff[0:1, 0:1] / (cp[3] * e_fp),
        f4[0:1, 2:3] * ff[0:1, 1:2] / (cp[4] * e_fp),
        f4[0:1, 3:4] * ff[0:1, 0:1] / (cp[5] * e_st),
        f4[0:1, 3:4] * ff[0:1, 1:2] / (cp[6] * e_st),
        jnp.zeros((1, _P), _f32),
    ]
    b = jnp.concatenate(rows, axis=0)
    beta[...] = b
    betaT[...] = b.T


def _combine_body(*refs):
    us_t = refs[0:7]
    us = refs[7:14]
    beta = refs[14]
    betaT = refs[15]
    unnorm = refs[16]
    d = refs[17]
    i = pl.program_id(0)
    bT = betaT[...]
    acc = jnp.zeros((_T, _P), _f32)
    for k, (u_t, u) in enumerate(zip(us_t, us)):
        m = _thresh(_dot_nt(u_t[...], u[...]))
        bj = beta[k:k + 1, :]
        bi = bT[:, k:k + 1]
        acc = acc + m * (bi + bj)
    unnorm[...] = acc
    dpart = jnp.concatenate(
        [jnp.sum(acc, axis=0, keepdims=True), jnp.zeros((7, _P), _f32)], axis=0)

    @pl.when(i == 0)
    def _():
        d[...] = dpart

    @pl.when(i > 0)
    def _():
        d[...] = d[...] + dpart


def _fin1_body(un, d, h1, b1, adj, x1):
    dr = jnp.maximum(d[0:1, :], 1e-12)
    a = un[...] / dr
    adj[...] = a
    x1[...] = jax.nn.relu(_dot(a, h1[...]) + b1[...])


def _fin2_body(adj, x1, w2, b2, out):
    h2 = _dot(x1[...], w2[...])
    y = _dot(adj[...], h2) + b2[...]
    y = y - jnp.max(y, axis=1, keepdims=True)
    out[...] = y - jnp.log(jnp.sum(jnp.exp(y), axis=1, keepdims=True))


def _full(shape):
    nd = len(shape)
    return pl.BlockSpec(shape, lambda i: (0,) * nd)


def kernel(features, adj_ori, mp_emb_mp1, mp_emb_mp2, enc_W, enc_b,
           fgg_origin_w, fp_origin_w, sgg_w, fgg_topo_pa_w, fgg_topo_ps_w,
           W_topo_pa, b_topo_pa, W_topo_ps, b_topo_ps, sg_agg_w, f_agg_f_w,
           f_agg_w, gcn_W1, gcn_b1, gcn_W2, gcn_b2):
    f_p = features[0:_P]
    f_a = features[_P:_P + _A]
    f_s = features[_P + _A:_N]
    mp1 = mp_emb_mp1[0:_P]
    mp2 = mp_emb_mp2[0:_P]
    sg1 = sgg_w[0]
    sg2 = sgg_w[1]
    bpa = b_topo_pa.reshape(1, _H)
    bps = b_topo_ps.reshape(1, _H)
    b1 = gcn_b1.reshape(1, _H)
    b2 = gcn_b2.reshape(1, 4)
    sgw = sg_agg_w.reshape(1, 2)
    ffw = f_agg_f_w.reshape(1, 2)
    fw = f_agg_w.reshape(1, 4)

    sd = jax.ShapeDtypeStruct
    u_t, u_1, u_2, u_ra, u_rs = pl.pallas_call(
        _prep_u_body,
        grid=(1,),
        in_specs=[_full((_P, _D)), _full((_A, _D)), _full((_S, _D)),
                  _full((_P, _H)), _full((_P, _H)), _full((2, _D)),
                  _full((2, _H)), _full((2, _H))],
        out_specs=[_full((_P, 2 * _D)), _full((_P, _D)), _full((_P, _D)),
                   _full((_A, 2 * _D)), _full((_S, 2 * _D))],
        out_shape=[sd((_P, 2 * _D), _bf16), sd((_P, _D), _bf16),
                   sd((_P, _D), _bf16), sd((_A, 2 * _D), _bf16),
                   sd((_S, 2 * _D), _bf16)],
    )(f_p, f_a, f_s, mp1, mp2, fgg_origin_w, sg1, sg2)

    sw_a = pl.pallas_call(
        _sw_body,
        grid=(_A // _T,),
        in_specs=[pl.BlockSpec((_T, 2 * _D), lambda i: (i, 0)),
                  _full((_A, 2 * _D)), _full((_A, _H))],
        out_specs=pl.BlockSpec((_T, _H), lambda i: (i, 0)),
        out_shape=sd((_A, _H), _f32),
    )(u_ra, u_ra, W_topo_pa)

    sw_s = pl.pallas_call(
        _sw_body,
        grid=(1,),
        in_specs=[_full((_S, 2 * _D)), _full((_S, 2 * _D)), _full((_S, _H))],
        out_specs=_full((_S, _H)),
        out_shape=sd((_S, _H), _f32),
    )(u_rs, u_rs, W_topo_ps)

    (u_fa, u_fs, u_sa, u_ss, h1) = pl.pallas_call(
        _prop_body,
        grid=(_P // _PT,),
        in_specs=[
            pl.BlockSpec((_PT, _A), lambda i: (i, _P // _A)),
            pl.BlockSpec((_PT, _S), lambda i: (i, (_P + _A) // _S)),
            _full((_A, _D)), _full((_S, _D)),
            _full((_A, _H)), _full((_S, _H)),
            pl.BlockSpec((_PT, _D), lambda i: (i, 0)),
            _full((_D, _H)),
            _full((2, _D)), _full((2, _H)), _full((2, _H)),
            _full((1, _H)), _full((1, _H)),
        ],
        out_specs=[
            pl.BlockSpec((_PT, 2 * _D), lambda i: (i, 0)),
            pl.BlockSpec((_PT, 2 * _D), lambda i: (i, 0)),
            pl.BlockSpec((_PT, _D), lambda i: (i, 0)),
            pl.BlockSpec((_PT, _D), lambda i: (i, 0)),
            pl.BlockSpec((_PT, _H), lambda i: (i, 0)),
        ],
        out_shape=[sd((_P, 2 * _D), _bf16), sd((_P, 2 * _D), _bf16),
                   sd((_P, _D), _bf16), sd((_P, _D), _bf16),
                   sd((_P, _H), _f32)],
    )(adj_ori, adj_ori, f_a, f_s, sw_a, sw_s, f_p, gcn_W1,
      fp_origin_w, fgg_topo_pa_w, fgg_topo_ps_w, bpa, bps)

    us = [u_t, u_1, u_2, u_fa, u_fs, u_sa, u_ss]
    ut_specs = [pl.BlockSpec((_T, u.shape[1]), lambda i: (i, 0)) for u in us]
    uf_specs = [_full(u.shape) for u in us]

    c = pl.pallas_call(
        _colsum_body,
        grid=(_P // _T,),
        in_specs=ut_specs + uf_specs,
        out_specs=_full((8, _P)),
        out_shape=sd((8, _P), _f32),
    )(*us, *us)

    beta, betaT = pl.pallas_call(
        _beta_body,
        grid=(1,),
        in_specs=[_full((8, _P)), _full((1, 2)), _full((1, 2)), _full((1, 4))],
        out_specs=[_full((8, _P)), _full((_P, 8))],
        out_shape=[sd((8, _P), _f32), sd((_P, 8), _f32)],
    )(c, sgw, ffw, fw)

    unnorm, dvec = pl.pallas_call(
        _combine_body,
        grid=(_P // _T,),
        in_specs=ut_specs + uf_specs + [
            _full((8, _P)), pl.BlockSpec((_T, 8), lambda i: (i, 0))],
        out_specs=[pl.BlockSpec((_T, _P), lambda i: (i, 0)), _full((8, _P))],
        out_shape=[sd((_P, _P), _f32), sd((8, _P), _f32)],
    )(*us, *us, beta, betaT)

    new_adj, x1 = pl.pallas_call(
        _fin1_body,
        grid=(_P // _T,),
        in_specs=[pl.BlockSpec((_T, _P), lambda i: (i, 0)), _full((8, _P)),
                  _full((_P, _H)), _full((1, _H))],
        out_specs=[pl.BlockSpec((_T, _P), lambda i: (i, 0)),
                   pl.BlockSpec((_T, _H), lambda i: (i, 0))],
        out_shape=[sd((_P, _P), _f32), sd((_P, _H), _f32)],
    )(unnorm, dvec, h1, b1)

    logits = pl.pallas_call(
        _fin2_body,
        grid=(_P // _T,),
        in_specs=[pl.BlockSpec((_T, _P), lambda i: (i, 0)), _full((_P, _H)),
                  _full((_H, 4)), _full((1, 4))],
        out_specs=pl.BlockSpec((_T, 4), lambda i: (i, 0)),
        out_shape=sd((_P, 4), _f32),
    )(new_adj, x1, gcn_W2, b2)

    return logits, new_adj


# speedup vs baseline: 2.0790x; 1.0413x over previous
import jax
import jax.numpy as jnp
from jax.experimental import pallas as pl
from jax.experimental.pallas import tpu as pltpu

_P, _A, _S = 2048, 2048, 512
_N = _P + _A + _S
_D = 128
_H = 64
_TH = 0.2
_T = 1024
_PT = 512

_f32 = jnp.float32
_bf16 = jnp.bfloat16


def _dot(a, b):
    return jax.lax.dot_general(a.astype(_bf16), b.astype(_bf16),
                               (((1,), (0,)), ((), ())),
                               preferred_element_type=_f32)


def _dot_nt(a, b):
    return jax.lax.dot_general(a.astype(_bf16), b.astype(_bf16),
                               (((1,), (1,)), ((), ())),
                               preferred_element_type=_f32)


def _unorm(x, m):
    y = x * m
    n = jnp.sqrt(jnp.sum(y * y, axis=1, keepdims=True))
    return y / jnp.maximum(n, 1e-8)


def _u2(x, mw):
    return jnp.concatenate(
        [_unorm(x, mw[0:1]), _unorm(x, mw[1:2])], axis=1).astype(_bf16)


def _thresh(g):
    s = 0.5 * g
    return jnp.where(s < _TH, jnp.zeros_like(s), s)


def _sm(v):
    e = jnp.exp(v - jnp.max(v, axis=1, keepdims=True))
    return e / jnp.sum(e, axis=1, keepdims=True)


def _prep_u_body(fp, fa, fs, mp1, mp2, fgg, sg1, sg2,
                 ut, u1, u2, ura, urs):
    ut[...] = _u2(fp[...], fgg[...])
    u1[...] = _u2(mp1[...], sg1[...])
    u2[...] = _u2(mp2[...], sg2[...])
    ura[...] = _u2(fa[...], fgg[...])
    urs[...] = _u2(fs[...], fgg[...])


def _sw_body(u_t, u, w, out):
    m = _thresh(_dot_nt(u_t[...], u[...]))
    out[...] = _dot(m, w[...])


def _prop_body(adj_a, adj_s, fa, fs, swa, sws, fp_t, gw1,
               fpw, tpaw, tpsw, bpa, bps,
               ufa, ufs, usa, uss, h1):
    aa = adj_a[...].astype(_bf16)
    as_ = adj_s[...].astype(_bf16)
    prop_a = jax.lax.dot_general(aa, fa[...].astype(_bf16),
                                 (((1,), (0,)), ((), ())),
                                 preferred_element_type=_f32)
    ufa[...] = _u2(prop_a, fpw[...])
    prop_s = jax.lax.dot_general(as_, fs[...].astype(_bf16),
                                 (((1,), (0,)), ((), ())),
                                 preferred_element_type=_f32)
    ufs[...] = _u2(prop_s, fpw[...])
    hid_a = jax.lax.dot_general(aa, swa[...].astype(_bf16),
                                (((1,), (0,)), ((), ())),
                                preferred_element_type=_f32) + bpa[...]
    usa[...] = _u2(hid_a, tpaw[...])
    hid_s = jax.lax.dot_general(as_, sws[...].astype(_bf16),
                                (((1,), (0,)), ((), ())),
                                preferred_element_type=_f32) + bps[...]
    uss[...] = _u2(hid_s, tpsw[...])
    h1[...] = _dot(fp_t[...], gw1[...])


def _colsum_body(*refs):
    us_t = refs[0:7]
    us = refs[7:14]
    c = refs[14]
    i = pl.program_id(0)
    rows = []
    for u_t, u in zip(us_t, us):
        m = _thresh(_dot_nt(u_t[...], u[...]))
        rows.append(jnp.sum(m, axis=0, keepdims=True))
    rows.append(jnp.zeros((1, _P), _f32))
    blk = jnp.concatenate(rows, axis=0)

    @pl.when(i == 0)
    def _():
        c[...] = blk

    @pl.when(i > 0)
    def _():
        c[...] = c[...] + blk


def _beta_body(c, sgw, ffw, fw, beta, betaT):
    sg = _sm(sgw[...])
    ff = _sm(ffw[...])
    f4 = _sm(fw[...])
    cs = [c[k:k + 1, :] for k in range(7)]
    cp = [jnp.maximum(x, 1e-12) for x in cs]
    t = [(x > 0.0).astype(_f32) for x in cs]
    eps = jnp.float32(1e-12)

    e_t = jnp.maximum(t[0], eps)
    e_sem = jnp.maximum(sg[0:1, 0:1] * t[1] + sg[0:1, 1:2] * t[2], eps)
    e_fp = jnp.maximum(ff[0:1, 0:1] * t[3] + ff[0:1, 1:2] * t[4], eps)
    e_st = jnp.maximum(ff[0:1, 0:1] * t[5] + ff[0:1, 1:2] * t[6], eps)

    rows = [
        f4[0:1, 0:1] / (cp[0] * e_t),
        f4[0:1, 1:2] * sg[0:1, 0:1] / (cp[1] * e_sem),
        f4[0:1, 1:2] * sg[0:1, 1:2] / (cp[2] * e_sem),
        f4[0:1, 2:3] * ff[0:1, 0:1] / (cp[3] * e_fp),
        f4[0:1, 2:3] * ff[0:1, 1:2] / (cp[4] * e_fp),
        f4[0:1, 3:4] * ff[0:1, 0:1] / (cp[5] * e_st),
        f4[0:1, 3:4] * ff[0:1, 1:2] / (cp[6] * e_st),
        jnp.zeros((1, _P), _f32),
    ]
    b = jnp.concatenate(rows, axis=0)
    beta[...] = b
    betaT[...] = b.T


def _combine_body(*refs):
    us_t = refs[0:7]
    us = refs[7:14]
    beta = refs[14]
    betaT = refs[15]
    unnorm = refs[16]
    d = refs[17]
    i = pl.program_id(0)
    bT = betaT[...]
    acc = jnp.zeros((_T, _P), _f32)
    for k, (u_t, u) in enumerate(zip(us_t, us)):
        m = _thresh(_dot_nt(u_t[...], u[...]))
        bj = beta[k:k + 1, :]
        bi = bT[:, k:k + 1]
        acc = acc + m * (bi + bj)
    unnorm[...] = acc
    dpart = jnp.concatenate(
        [jnp.sum(acc, axis=0, keepdims=True), jnp.zeros((7, _P), _f32)], axis=0)

    @pl.when(i == 0)
    def _():
        d[...] = dpart

    @pl.when(i > 0)
    def _():
        d[...] = d[...] + dpart


def _fin1_body(un, d, h1, b1, adj, x1):
    dr = jnp.maximum(d[0:1, :], 1e-12)
    a = un[...] / dr
    adj[...] = a
    x1[...] = jax.nn.relu(_dot(a, h1[...]) + b1[...])


def _fin2_body(adj, x1, w2, b2, out):
    h2 = _dot(x1[...], w2[...])
    y = _dot(adj[...], h2) + b2[...]
    y = y - jnp.max(y, axis=1, keepdims=True)
    out[...] = y - jnp.log(jnp.sum(jnp.exp(y), axis=1, keepdims=True))


def _full(shape):
    nd = len(shape)
    return pl.BlockSpec(shape, lambda i: (0,) * nd)


def kernel(features, adj_ori, mp_emb_mp1, mp_emb_mp2, enc_W, enc_b,
           fgg_origin_w, fp_origin_w, sgg_w, fgg_topo_pa_w, fgg_topo_ps_w,
           W_topo_pa, b_topo_pa, W_topo_ps, b_topo_ps, sg_agg_w, f_agg_f_w,
           f_agg_w, gcn_W1, gcn_b1, gcn_W2, gcn_b2):
    f_p = features[0:_P]
    f_a = features[_P:_P + _A]
    f_s = features[_P + _A:_N]
    mp1 = mp_emb_mp1[0:_P]
    mp2 = mp_emb_mp2[0:_P]
    sg1 = sgg_w[0]
    sg2 = sgg_w[1]
    bpa = b_topo_pa.reshape(1, _H)
    bps = b_topo_ps.reshape(1, _H)
    b1 = gcn_b1.reshape(1, _H)
    b2 = gcn_b2.reshape(1, 4)
    sgw = sg_agg_w.reshape(1, 2)
    ffw = f_agg_f_w.reshape(1, 2)
    fw = f_agg_w.reshape(1, 4)

    sd = jax.ShapeDtypeStruct
    u_t, u_1, u_2, u_ra, u_rs = pl.pallas_call(
        _prep_u_body,
        grid=(1,),
        in_specs=[_full((_P, _D)), _full((_A, _D)), _full((_S, _D)),
                  _full((_P, _H)), _full((_P, _H)), _full((2, _D)),
                  _full((2, _H)), _full((2, _H))],
        out_specs=[_full((_P, 2 * _D)), _full((_P, _D)), _full((_P, _D)),
                   _full((_A, 2 * _D)), _full((_S, 2 * _D))],
        out_shape=[sd((_P, 2 * _D), _bf16), sd((_P, _D), _bf16),
                   sd((_P, _D), _bf16), sd((_A, 2 * _D), _bf16),
                   sd((_S, 2 * _D), _bf16)],
    )(f_p, f_a, f_s, mp1, mp2, fgg_origin_w, sg1, sg2)

    sw_a = pl.pallas_call(
        _sw_body,
        grid=(_A // _T,),
        in_specs=[pl.BlockSpec((_T, 2 * _D), lambda i: (i, 0)),
                  _full((_A, 2 * _D)), _full((_A, _H))],
        out_specs=pl.BlockSpec((_T, _H), lambda i: (i, 0)),
        out_shape=sd((_A, _H), _f32),
    )(u_ra, u_ra, W_topo_pa)

    sw_s = pl.pallas_call(
        _sw_body,
        grid=(1,),
        in_specs=[_full((_S, 2 * _D)), _full((_S, 2 * _D)), _full((_S, _H))],
        out_specs=_full((_S, _H)),
        out_shape=sd((_S, _H), _f32),
    )(u_rs, u_rs, W_topo_ps)

    (u_fa, u_fs, u_sa, u_ss, h1) = pl.pallas_call(
        _prop_body,
        grid=(_P // _PT,),
        in_specs=[
            pl.BlockSpec((_PT, _A), lambda i: (i, _P // _A)),
            pl.BlockSpec((_PT, _S), lambda i: (i, (_P + _A) // _S)),
            _full((_A, _D)), _full((_S, _D)),
            _full((_A, _H)), _full((_S, _H)),
            pl.BlockSpec((_PT, _D), lambda i: (i, 0)),
            _full((_D, _H)),
            _full((2, _D)), _full((2, _H)), _full((2, _H)),
            _full((1, _H)), _full((1, _H)),
        ],
        out_specs=[
            pl.BlockSpec((_PT, 2 * _D), lambda i: (i, 0)),
            pl.BlockSpec((_PT, 2 * _D), lambda i: (i, 0)),
            pl.BlockSpec((_PT, _D), lambda i: (i, 0)),
            pl.BlockSpec((_PT, _D), lambda i: (i, 0)),
            pl.BlockSpec((_PT, _H), lambda i: (i, 0)),
        ],
        out_shape=[sd((_P, 2 * _D), _bf16), sd((_P, 2 * _D), _bf16),
                   sd((_P, _D), _bf16), sd((_P, _D), _bf16),
                   sd((_P, _H), _f32)],
    )(adj_ori, adj_ori, f_a, f_s, sw_a, sw_s, f_p, gcn_W1,
      fp_origin_w, fgg_topo_pa_w, fgg_topo_ps_w, bpa, bps)

    us = [u_t, u_1, u_2, u_fa, u_fs, u_sa, u_ss]
    ut_specs = [pl.BlockSpec((_T, u.shape[1]), lambda i: (i, 0)) for u in us]
    uf_specs = [_full(u.shape) for u in us]

    c = pl.pallas_call(
        _colsum_body,
        grid=(_P // _T,),
        in_specs=ut_specs + uf_specs,
        out_specs=_full((8, _P)),
        out_shape=sd((8, _P), _f32),
    )(*us, *us)

    beta, betaT = pl.pallas_call(
        _beta_body,
        grid=(1,),
        in_specs=[_full((8, _P)), _full((1, 2)), _full((1, 2)), _full((1, 4))],
        out_specs=[_full((8, _P)), _full((_P, 8))],
        out_shape=[sd((8, _P), _f32), sd((_P, 8), _f32)],
    )(c, sgw, ffw, fw)

    unnorm, dvec = pl.pallas_call(
        _combine_body,
        grid=(_P // _T,),
        in_specs=ut_specs + uf_specs + [
            _full((8, _P)), pl.BlockSpec((_T, 8), lambda i: (i, 0))],
        out_specs=[pl.BlockSpec((_T, _P), lambda i: (i, 0)), _full((8, _P))],
        out_shape=[sd((_P, _P), _f32), sd((8, _P), _f32)],
    )(*us, *us, beta, betaT)

    new_adj, x1 = pl.pallas_call(
        _fin1_body,
        grid=(_P // _T,),
        in_specs=[pl.BlockSpec((_T, _P), lambda i: (i, 0)), _full((8, _P)),
                  _full((_P, _H)), _full((1, _H))],
        out_specs=[pl.BlockSpec((_T, _P), lambda i: (i, 0)),
                   pl.BlockSpec((_T, _H), lambda i: (i, 0))],
        out_shape=[sd((_P, _P), _f32), sd((_P, _H), _f32)],
    )(unnorm, dvec, h1, b1)

    logits = pl.pallas_call(
        _fin2_body,
        grid=(_P // _T,),
        in_specs=[pl.BlockSpec((_T, _P), lambda i: (i, 0)), _full((_P, _H)),
                  _full((_H, 4)), _full((1, 4))],
        out_specs=pl.BlockSpec((_T, 4), lambda i: (i, 0)),
        out_shape=sd((_P, 4), _f32),
    )(new_adj, x1, gcn_W2, b2)

    return logits, new_adj
